# initial kernel scaffold (unmeasured)
import jax
import jax.numpy as jnp
from jax import lax
from jax.experimental import pallas as pl
from jax.experimental.pallas import tpu as pltpu

N_DEV = 8
B = 8
H = 8
D = 64
BH = B * H
COLS = 128


def kernel(Q, K, V):
    def body(q_ref, k_ref, v_ref, out_ref, comm_ref, send_sems, recv_sems):
        my = lax.axis_index("i")

        q = q_ref[:, 0, :, :]
        k = k_ref[...]
        v = v_ref[...]
        scale = D ** -0.5
        s = jnp.sum(q[:, None, :, :] * k, axis=-1) * scale
        m = jnp.max(s, axis=1)
        p = jnp.exp(s - m[:, None, :])
        l = jnp.sum(p, axis=1)
        o = jnp.sum(p[:, :, :, None] * v, axis=1)

        comm_ref[0, :, 0:D] = o.reshape(BH, D)
        comm_ref[0, :, D:D + 1] = m.reshape(BH, 1)
        comm_ref[0, :, D + 1:D + 2] = l.reshape(BH, 1)

        barrier = pltpu.get_barrier_semaphore()
        for d in range(1, N_DEV):
            peer = lax.rem(my + d, N_DEV)
            pl.semaphore_signal(
                barrier, inc=1,
                device_id=(peer,), device_id_type=pl.DeviceIdType.MESH,
            )
        pl.semaphore_wait(barrier, N_DEV - 1)

        rdmas = []
        for d in range(1, N_DEV):
            peer = lax.rem(my + d, N_DEV)
            rdma = pltpu.make_async_remote_copy(
                src_ref=comm_ref.at[0],
                dst_ref=comm_ref.at[d],
                send_sem=send_sems.at[d],
                recv_sem=recv_sems.at[d],
                device_id=(peer,),
                device_id_type=pl.DeviceIdType.MESH,
            )
            rdma.start()
            rdmas.append(rdma)
        for rdma in rdmas:
            rdma.wait_recv()
        for rdma in rdmas:
            rdma.wait_send()

        data = comm_ref[...]
        o_all = data[:, :, 0:D]
        m_all = data[:, :, D]
        l_all = data[:, :, D + 1]
        m_g = jnp.max(m_all, axis=0)
        sc = jnp.exp(m_all - m_g[None, :])
        l_g = jnp.sum(l_all * sc, axis=0)
        o_g = jnp.sum(o_all * sc[:, :, None], axis=0) / l_g[:, None]
        out_ref[...] = o_g.reshape(B, 1, H, D)

    return pl.pallas_call(
        body,
        out_shape=jax.ShapeDtypeStruct((B, 1, H, D), jnp.float32),
        in_specs=[
            pl.BlockSpec(memory_space=pltpu.VMEM),
            pl.BlockSpec(memory_space=pltpu.VMEM),
            pl.BlockSpec(memory_space=pltpu.VMEM),
        ],
        out_specs=pl.BlockSpec(memory_space=pltpu.VMEM),
        scratch_shapes=[
            pltpu.VMEM((N_DEV, BH, COLS), jnp.float32),
            pltpu.SemaphoreType.DMA((N_DEV,)),
            pltpu.SemaphoreType.DMA((N_DEV,)),
        ],
        compiler_params=pltpu.CompilerParams(collective_id=0),
    )(Q, K, V)


# baseline (device time: 44105 ns/iter reference)
import jax
import jax.numpy as jnp
from jax import lax
from jax.experimental import pallas as pl
from jax.experimental.pallas import tpu as pltpu

N_DEV = 8
B = 8
H = 8
D = 64
BH = B * H
COLS = 128


def kernel(Q, K, V):
    def body(q_ref, k_ref, v_ref, out_ref, comm_ref, send_sems, recv_sems):
        my = lax.axis_index("i")

        q = q_ref[:, 0, :, :]
        k = k_ref[...]
        v = v_ref[...]
        scale = D ** -0.5
        s = jnp.sum(q[:, None, :, :] * k, axis=-1) * scale
        m = jnp.max(s, axis=1)
        p = jnp.exp(s - m[:, None, :])
        l = jnp.sum(p, axis=1)
        o = jnp.sum(p[:, :, :, None] * v, axis=1)

        comm_ref[0, :, :, 0:D] = o
        comm_ref[0, :, :, D:D + 1] = m[:, :, None]
        comm_ref[0, :, :, D + 1:D + 2] = l[:, :, None]

        barrier = pltpu.get_barrier_semaphore()
        for d in range(1, N_DEV):
            peer = lax.rem(my + d, N_DEV)
            pl.semaphore_signal(
                barrier, inc=1,
                device_id=(peer,), device_id_type=pl.DeviceIdType.MESH,
            )
        pl.semaphore_wait(barrier, N_DEV - 1)

        rdmas = []
        for d in range(1, N_DEV):
            peer = lax.rem(my + d, N_DEV)
            rdma = pltpu.make_async_remote_copy(
                src_ref=comm_ref.at[0],
                dst_ref=comm_ref.at[d],
                send_sem=send_sems.at[d],
                recv_sem=recv_sems.at[d],
                device_id=(peer,),
                device_id_type=pl.DeviceIdType.MESH,
            )
            rdma.start()
            rdmas.append(rdma)
        for rdma in rdmas:
            rdma.wait_recv()
        for rdma in rdmas:
            rdma.wait_send()

        data = comm_ref[...]
        o_all = data[:, :, :, 0:D]
        m_all = data[:, :, :, D]
        l_all = data[:, :, :, D + 1]
        m_g = jnp.max(m_all, axis=0)
        sc = jnp.exp(m_all - m_g[None, :, :])
        l_g = jnp.sum(l_all * sc, axis=0)
        o_g = jnp.sum(o_all * sc[:, :, :, None], axis=0)
        o_g = o_g / l_g[:, :, None]
        out_ref[...] = o_g[:, None, :, :]

    return pl.pallas_call(
        body,
        out_shape=jax.ShapeDtypeStruct((B, 1, H, D), jnp.float32),
        in_specs=[
            pl.BlockSpec(memory_space=pltpu.VMEM),
            pl.BlockSpec(memory_space=pltpu.VMEM),
            pl.BlockSpec(memory_space=pltpu.VMEM),
        ],
        out_specs=pl.BlockSpec(memory_space=pltpu.VMEM),
        scratch_shapes=[
            pltpu.VMEM((N_DEV, B, H, COLS), jnp.float32),
            pltpu.SemaphoreType.DMA((N_DEV,)),
            pltpu.SemaphoreType.DMA((N_DEV,)),
        ],
        compiler_params=pltpu.CompilerParams(collective_id=0),
    )(Q, K, V)


# device time: 17706 ns/iter; 2.4910x vs baseline; 2.4910x over previous
import jax
import jax.numpy as jnp
from jax import lax
from jax.experimental import pallas as pl
from jax.experimental.pallas import tpu as pltpu

N_DEV = 8
B = 8
H = 8
D = 64
COLS = 128


def kernel(Q, K, V):
    Kt = jnp.transpose(K, (0, 2, 3, 1))
    Vt = jnp.transpose(V, (0, 2, 3, 1))

    def body(q_ref, k_ref, v_ref, out_ref, comm_ref, send_sems, recv_sems):
        my = lax.axis_index("i")

        q = q_ref[:, 0, :, :]
        kt = k_ref[...]
        vt = v_ref[...]
        scale = D ** -0.5
        s = jnp.sum(q[:, :, :, None] * kt, axis=2) * scale
        m = jnp.max(s, axis=-1)
        p = jnp.exp(s - m[:, :, None])
        l = jnp.sum(p, axis=-1)
        o = jnp.sum(p[:, :, None, :] * vt, axis=-1)

        comm_ref[0, :, :, 0:D] = o
        comm_ref[0, :, :, D:D + 1] = m[:, :, None]
        comm_ref[0, :, :, D + 1:D + 2] = l[:, :, None]

        barrier = pltpu.get_barrier_semaphore()
        for d in range(1, N_DEV):
            peer = lax.rem(my + d, N_DEV)
            pl.semaphore_signal(
                barrier, inc=1,
                device_id=(peer,), device_id_type=pl.DeviceIdType.MESH,
            )
        pl.semaphore_wait(barrier, N_DEV - 1)

        rdmas = []
        for d in range(1, N_DEV):
            peer = lax.rem(my + d, N_DEV)
            rdma = pltpu.make_async_remote_copy(
                src_ref=comm_ref.at[0],
                dst_ref=comm_ref.at[d],
                send_sem=send_sems.at[d],
                recv_sem=recv_sems.at[d],
                device_id=(peer,),
                device_id_type=pl.DeviceIdType.MESH,
            )
            rdma.start()
            rdmas.append(rdma)
        for rdma in rdmas:
            rdma.wait_recv()
        for rdma in rdmas:
            rdma.wait_send()

        data = comm_ref[...]
        o_all = data[:, :, :, 0:D]
        m_all = data[:, :, :, D]
        l_all = data[:, :, :, D + 1]
        m_g = jnp.max(m_all, axis=0)
        sc = jnp.exp(m_all - m_g[None, :, :])
        l_g = jnp.sum(l_all * sc, axis=0)
        o_g = jnp.sum(o_all * sc[:, :, :, None], axis=0)
        o_g = o_g / l_g[:, :, None]
        out_ref[...] = o_g[:, None, :, :]

    return pl.pallas_call(
        body,
        out_shape=jax.ShapeDtypeStruct((B, 1, H, D), jnp.float32),
        in_specs=[
            pl.BlockSpec(memory_space=pltpu.MemorySpace.VMEM),
            pl.BlockSpec(memory_space=pltpu.MemorySpace.VMEM),
            pl.BlockSpec(memory_space=pltpu.MemorySpace.VMEM),
        ],
        out_specs=pl.BlockSpec(memory_space=pltpu.MemorySpace.VMEM),
        scratch_shapes=[
            pltpu.VMEM((N_DEV, B, H, COLS), jnp.float32),
            pltpu.SemaphoreType.DMA((N_DEV,)),
            pltpu.SemaphoreType.DMA((N_DEV,)),
        ],
        compiler_params=pltpu.CompilerParams(
            collective_id=0,
            vmem_limit_bytes=100 * 1024 * 1024,
        ),
    )(Q, Kt, Vt)


# device time: 16410 ns/iter; 2.6877x vs baseline; 1.0790x over previous
import jax
import jax.numpy as jnp
from jax import lax
from jax.experimental import pallas as pl
from jax.experimental.pallas import tpu as pltpu

N_DEV = 8
B = 8
H = 8
D = 64
KV = 512
COLS = 128
CHUNKS = 8
CB = B // CHUNKS


def kernel(Q, K, V):
    Kt = jnp.transpose(K, (0, 2, 3, 1))
    Vt = jnp.transpose(V, (0, 2, 3, 1))

    def body(q_ref, k_hbm, v_hbm, out_ref,
             k_scr, v_scr, comm_ref, copy_sems, send_sems, recv_sems):
        my = lax.axis_index("i")
        scale = D ** -0.5

        copies = []
        for c in range(CHUNKS):
            bs = pl.ds(c * CB, CB)
            ck = pltpu.make_async_copy(
                k_hbm.at[bs], k_scr.at[bs], copy_sems.at[2 * c])
            cv = pltpu.make_async_copy(
                v_hbm.at[bs], v_scr.at[bs], copy_sems.at[2 * c + 1])
            ck.start()
            cv.start()
            copies.append((ck, cv))

        barrier = pltpu.get_barrier_semaphore()
        for d in range(1, N_DEV):
            peer = lax.rem(my + d, N_DEV)
            pl.semaphore_signal(
                barrier, inc=1,
                device_id=(peer,), device_id_type=pl.DeviceIdType.MESH,
            )

        for c in range(CHUNKS):
            ck, cv = copies[c]
            ck.wait()
            cv.wait()
            bs = pl.ds(c * CB, CB)
            q = q_ref[bs, 0, :, :]
            kt = k_scr[bs, :, :, :]
            vt = v_scr[bs, :, :, :]
            s = jnp.sum(q[:, :, :, None] * kt, axis=2) * scale
            m = jnp.max(s, axis=-1)
            p = jnp.exp(s - m[:, :, None])
            l = jnp.sum(p, axis=-1)
            o = jnp.sum(p[:, :, None, :] * vt, axis=-1)
            comm_ref[0, bs, :, 0:D] = o
            comm_ref[0, bs, :, D:D + 1] = m[:, :, None]
            comm_ref[0, bs, :, D + 1:D + 2] = l[:, :, None]

        pl.semaphore_wait(barrier, N_DEV - 1)

        rdmas = []
        for d in range(1, N_DEV):
            peer = lax.rem(my + d, N_DEV)
            rdma = pltpu.make_async_remote_copy(
                src_ref=comm_ref.at[0],
                dst_ref=comm_ref.at[d],
                send_sem=send_sems.at[d],
                recv_sem=recv_sems.at[d],
                device_id=(peer,),
                device_id_type=pl.DeviceIdType.MESH,
            )
            rdma.start()
            rdmas.append(rdma)
        for rdma in rdmas:
            rdma.wait_recv()
        for rdma in rdmas:
            rdma.wait_send()

        data = comm_ref[...]
        o_all = data[:, :, :, 0:D]
        m_all = data[:, :, :, D]
        l_all = data[:, :, :, D + 1]
        m_g = jnp.max(m_all, axis=0)
        sc = jnp.exp(m_all - m_g[None, :, :])
        l_g = jnp.sum(l_all * sc, axis=0)
        o_g = jnp.sum(o_all * sc[:, :, :, None], axis=0)
        o_g = o_g / l_g[:, :, None]
        out_ref[...] = o_g[:, None, :, :]

    return pl.pallas_call(
        body,
        out_shape=jax.ShapeDtypeStruct((B, 1, H, D), jnp.float32),
        in_specs=[
            pl.BlockSpec(memory_space=pltpu.MemorySpace.VMEM),
            pl.BlockSpec(memory_space=pltpu.MemorySpace.HBM),
            pl.BlockSpec(memory_space=pltpu.MemorySpace.HBM),
        ],
        out_specs=pl.BlockSpec(memory_space=pltpu.MemorySpace.VMEM),
        scratch_shapes=[
            pltpu.VMEM((B, H, D, KV), jnp.float32),
            pltpu.VMEM((B, H, D, KV), jnp.float32),
            pltpu.VMEM((N_DEV, B, H, COLS), jnp.float32),
            pltpu.SemaphoreType.DMA((2 * CHUNKS,)),
            pltpu.SemaphoreType.DMA((N_DEV,)),
            pltpu.SemaphoreType.DMA((N_DEV,)),
        ],
        compiler_params=pltpu.CompilerParams(
            collective_id=0,
            vmem_limit_bytes=100 * 1024 * 1024,
        ),
    )(Q, Kt, Vt)


# device time: 16342 ns/iter; 2.6989x vs baseline; 1.0042x over previous
import jax
import jax.numpy as jnp
from jax import lax
from jax.experimental import pallas as pl
from jax.experimental.pallas import tpu as pltpu

N_DEV = 8
B = 8
H = 8
D = 64
KV = 512
COLS = 128
CHUNKS = 8
CB = B // CHUNKS
DEPTH = 4


def kernel(Q, K, V):
    Kt = jnp.transpose(K, (0, 2, 3, 1))
    Vt = jnp.transpose(V, (0, 2, 3, 1))

    def body(q_ref, k_hbm, v_hbm, out_ref,
             k_scr, v_scr, comm_ref, copy_sems, send_sems, recv_sems):
        my = lax.axis_index("i")
        scale = D ** -0.5

        copies = [None] * CHUNKS

        def start_chunk(j):
            bsj = pl.ds(j * CB, CB)
            ckj = pltpu.make_async_copy(
                k_hbm.at[bsj], k_scr.at[bsj], copy_sems.at[2 * j])
            cvj = pltpu.make_async_copy(
                v_hbm.at[bsj], v_scr.at[bsj], copy_sems.at[2 * j + 1])
            ckj.start()
            cvj.start()
            copies[j] = (ckj, cvj)

        for j in range(DEPTH):
            start_chunk(j)

        barrier = pltpu.get_barrier_semaphore()
        for d in range(1, N_DEV):
            peer = lax.rem(my + d, N_DEV)
            pl.semaphore_signal(
                barrier, inc=1,
                device_id=(peer,), device_id_type=pl.DeviceIdType.MESH,
            )

        for c in range(CHUNKS):
            ck, cv = copies[c]
            ck.wait()
            cv.wait()
            if c + DEPTH < CHUNKS:
                start_chunk(c + DEPTH)
            bs = pl.ds(c * CB, CB)
            q = q_ref[bs, 0, :, :]
            kt = k_scr[bs, :, :, :]
            vt = v_scr[bs, :, :, :]
            s = jnp.sum(q[:, :, :, None] * kt, axis=2) * scale
            m = jnp.max(s, axis=-1)
            p = jnp.exp(s - m[:, :, None])
            l = jnp.sum(p, axis=-1)
            o = jnp.sum(p[:, :, None, :] * vt, axis=-1)
            comm_ref[0, bs, :, 0:D] = o
            comm_ref[0, bs, :, D:D + 1] = m[:, :, None]
            comm_ref[0, bs, :, D + 1:D + 2] = l[:, :, None]

        pl.semaphore_wait(barrier, N_DEV - 1)

        rdmas = []
        for d in range(1, N_DEV):
            peer = lax.rem(my + d, N_DEV)
            rdma = pltpu.make_async_remote_copy(
                src_ref=comm_ref.at[0],
                dst_ref=comm_ref.at[d],
                send_sem=send_sems.at[d],
                recv_sem=recv_sems.at[d],
                device_id=(peer,),
                device_id_type=pl.DeviceIdType.MESH,
            )
            rdma.start()
            rdmas.append(rdma)
        for rdma in rdmas:
            rdma.wait_recv()
        for rdma in rdmas:
            rdma.wait_send()

        data = comm_ref[...]
        o_all = data[:, :, :, 0:D]
        m_all = data[:, :, :, D]
        l_all = data[:, :, :, D + 1]
        m_g = jnp.max(m_all, axis=0)
        sc = jnp.exp(m_all - m_g[None, :, :])
        l_g = jnp.sum(l_all * sc, axis=0)
        o_g = jnp.sum(o_all * sc[:, :, :, None], axis=0)
        o_g = o_g / l_g[:, :, None]
        out_ref[...] = o_g[:, None, :, :]

    return pl.pallas_call(
        body,
        out_shape=jax.ShapeDtypeStruct((B, 1, H, D), jnp.float32),
        in_specs=[
            pl.BlockSpec(memory_space=pltpu.MemorySpace.VMEM),
            pl.BlockSpec(memory_space=pltpu.MemorySpace.HBM),
            pl.BlockSpec(memory_space=pltpu.MemorySpace.HBM),
        ],
        out_specs=pl.BlockSpec(memory_space=pltpu.MemorySpace.VMEM),
        scratch_shapes=[
            pltpu.VMEM((B, H, D, KV), jnp.float32),
            pltpu.VMEM((B, H, D, KV), jnp.float32),
            pltpu.VMEM((N_DEV, B, H, COLS), jnp.float32),
            pltpu.SemaphoreType.DMA((2 * CHUNKS,)),
            pltpu.SemaphoreType.DMA((N_DEV,)),
            pltpu.SemaphoreType.DMA((N_DEV,)),
        ],
        compiler_params=pltpu.CompilerParams(
            collective_id=0,
            vmem_limit_bytes=100 * 1024 * 1024,
        ),
    )(Q, Kt, Vt)


# device time: 16171 ns/iter; 2.7274x vs baseline; 1.0106x over previous
import jax
import jax.numpy as jnp
from jax import lax
from jax.experimental import pallas as pl
from jax.experimental.pallas import tpu as pltpu

N_DEV = 8
B = 8
H = 8
D = 64
KV = 512
COLS = 128
CHUNKS = 8
CB = B // CHUNKS
DEPTH = 4
GROUPS = 2


def kernel(Q, K, V):
    Kt = jnp.transpose(K, (0, 2, 3, 1))
    Vt = jnp.transpose(V, (0, 2, 3, 1))

    def body(q_ref, k_hbm, v_hbm, out_ref,
             k_scr, v_scr, comm_ref, copy_sems, send_sems, recv_sems):
        my = lax.axis_index("i")
        scale = D ** -0.5

        copies = [None] * CHUNKS

        def start_chunk(j):
            bsj = pl.ds(j * CB, CB)
            ckj = pltpu.make_async_copy(
                k_hbm.at[bsj], k_scr.at[bsj], copy_sems.at[2 * j])
            cvj = pltpu.make_async_copy(
                v_hbm.at[bsj], v_scr.at[bsj], copy_sems.at[2 * j + 1])
            ckj.start()
            cvj.start()
            copies[j] = (ckj, cvj)

        for j in range(DEPTH):
            start_chunk(j)

        barrier = pltpu.get_barrier_semaphore()
        for d in range(1, N_DEV):
            peer = lax.rem(my + d, N_DEV)
            pl.semaphore_signal(
                barrier, inc=1,
                device_id=(peer,), device_id_type=pl.DeviceIdType.MESH,
            )

        GB = B // GROUPS
        rdmas = []

        def send_group(g):
            gbs = pl.ds(g * GB, GB)
            for d in range(1, N_DEV):
                peer = lax.rem(my + d, N_DEV)
                rdma = pltpu.make_async_remote_copy(
                    src_ref=comm_ref.at[0, gbs],
                    dst_ref=comm_ref.at[d, gbs],
                    send_sem=send_sems.at[g, d],
                    recv_sem=recv_sems.at[g, d],
                    device_id=(peer,),
                    device_id_type=pl.DeviceIdType.MESH,
                )
                rdma.start()
                rdmas.append(rdma)

        for c in range(CHUNKS):
            ck, cv = copies[c]
            ck.wait()
            cv.wait()
            if c + DEPTH < CHUNKS:
                start_chunk(c + DEPTH)
            bs = pl.ds(c * CB, CB)
            q = q_ref[bs, 0, :, :]
            kt = k_scr[bs, :, :, :]
            vt = v_scr[bs, :, :, :]
            s = jnp.sum(q[:, :, :, None] * kt, axis=2) * scale
            m = jnp.max(s, axis=-1)
            p = jnp.exp(s - m[:, :, None])
            l = jnp.sum(p, axis=-1)
            o = jnp.sum(p[:, :, None, :] * vt, axis=-1)
            comm_ref[0, bs, :, 0:D] = o
            comm_ref[0, bs, :, D:D + 1] = m[:, :, None]
            comm_ref[0, bs, :, D + 1:D + 2] = l[:, :, None]
            if (c + 1) % (CHUNKS // GROUPS) == 0:
                g = (c + 1) // (CHUNKS // GROUPS) - 1
                if g == 0:
                    pl.semaphore_wait(barrier, N_DEV - 1)
                send_group(g)

        for rdma in rdmas:
            rdma.wait_recv()
        for rdma in rdmas:
            rdma.wait_send()

        data = comm_ref[...]
        o_all = data[:, :, :, 0:D]
        m_all = data[:, :, :, D]
        l_all = data[:, :, :, D + 1]
        m_g = jnp.max(m_all, axis=0)
        sc = jnp.exp(m_all - m_g[None, :, :])
        l_g = jnp.sum(l_all * sc, axis=0)
        o_g = jnp.sum(o_all * sc[:, :, :, None], axis=0)
        o_g = o_g / l_g[:, :, None]
        out_ref[...] = o_g[:, None, :, :]

    return pl.pallas_call(
        body,
        out_shape=jax.ShapeDtypeStruct((B, 1, H, D), jnp.float32),
        in_specs=[
            pl.BlockSpec(memory_space=pltpu.MemorySpace.VMEM),
            pl.BlockSpec(memory_space=pltpu.MemorySpace.HBM),
            pl.BlockSpec(memory_space=pltpu.MemorySpace.HBM),
        ],
        out_specs=pl.BlockSpec(memory_space=pltpu.MemorySpace.VMEM),
        scratch_shapes=[
            pltpu.VMEM((B, H, D, KV), jnp.float32),
            pltpu.VMEM((B, H, D, KV), jnp.float32),
            pltpu.VMEM((N_DEV, B, H, COLS), jnp.float32),
            pltpu.SemaphoreType.DMA((2 * CHUNKS,)),
            pltpu.SemaphoreType.DMA((GROUPS, N_DEV)),
            pltpu.SemaphoreType.DMA((GROUPS, N_DEV)),
        ],
        compiler_params=pltpu.CompilerParams(
            collective_id=0,
            vmem_limit_bytes=100 * 1024 * 1024,
        ),
    )(Q, Kt, Vt)
